# baseline (speedup 1.0000x reference)
import functools

import jax
import jax.numpy as jnp
from jax import lax
from jax.experimental import pallas as pl
from jax.experimental.pallas import tpu as pltpu
from jax.experimental.pallas import tpu_sc as plsc

VOCAB = 1000
D = 1000
DPAD = 1024
BATCH = 4096
SEQ = 50
NW = 32
CD = 40
N_CHUNKS = D // CD
L = 16
NG = 128 // L
OBUF_BYTES = CD * 128 * 4

_mesh = plsc.VectorSubcoreMesh(core_axis_name="c", subcore_axis_name="s")


@functools.partial(
    pl.kernel,
    mesh=_mesh,
    out_type=jax.ShapeDtypeStruct((SEQ, D, BATCH), jnp.float32),
    compiler_params=pltpu.CompilerParams(use_tc_tiling_on_sc=True,
                                         needs_layout_passes=False),
    scratch_types=[
        pltpu.VMEM((SEQ, 128), jnp.int32),
        pltpu.VMEM((2, CD, DPAD), jnp.float32),
        pltpu.VMEM((2, CD, 128), jnp.float32),
        pltpu.SemaphoreType.DMA,
        pltpu.SemaphoreType.DMA,
        pltpu.SemaphoreType.DMA,
        pltpu.SemaphoreType.DMA,
    ],
)
def _emb_transposed(idx_hbm, tableT_hbm, out_hbm, idx_v, chunks, obufs,
                    g0, g1, o0, o1):
    gsems = [g0, g1]
    osems = [o0, o1]
    wid = lax.axis_index("s") * 2 + lax.axis_index("c")
    bcol = wid * 128
    pltpu.sync_copy(idx_hbm.at[:, pl.ds(bcol, 128)], idx_v)

    def chunk_src(c):
        return tableT_hbm.at[pl.ds(CD * c, CD), :]

    def out_dst(s, c):
        return out_hbm.at[s, pl.ds(CD * c, CD), pl.ds(bcol, 128)]

    def compute_tile(s, cbuf, obuf):
        cols = [idx_v[s, pl.ds(L * g, L)] for g in range(NG)]

        def drow(d, carry):
            dvec = jnp.full((L,), d, dtype=jnp.int32)
            for g in range(NG):
                obuf[d, pl.ds(L * g, L)] = plsc.load_gather(cbuf, [dvec, cols[g]])
            return carry

        lax.fori_loop(0, CD, drow, 0)

    def s_loop(c, cbuf, first):
        def s_pair(i, carry):
            for p in range(2):
                s = 2 * i + p

                @pl.when(jnp.logical_or(jnp.logical_not(first), i > 0))
                def _():
                    pltpu.make_async_copy(obufs.at[p], out_dst(0, 0),
                                          osems[p]).wait()
                compute_tile(s, cbuf, obufs.at[p])
                pltpu.async_copy(obufs.at[p], out_dst(s, c), osems[p])
            return carry

        lax.fori_loop(0, SEQ // 2, s_pair, 0)

    def wait_chunk(c, d):
        pltpu.make_async_copy(chunk_src(0), chunks.at[d], gsems[d]).wait()

    pltpu.async_copy(chunk_src(0), chunks.at[0], gsems[0])

    def chunk_pair(j, carry):
        c0 = 2 * j
        pltpu.async_copy(chunk_src(c0 + 1), chunks.at[1], gsems[1])
        wait_chunk(c0, 0)
        s_loop(c0, chunks.at[0], j == 0)
        pltpu.async_copy(chunk_src(c0 + 2), chunks.at[0], gsems[0])
        wait_chunk(c0 + 1, 1)
        s_loop(c0 + 1, chunks.at[1], jnp.bool_(False))
        return carry

    lax.fori_loop(0, N_CHUNKS // 2, chunk_pair, 0)
    wait_chunk(N_CHUNKS - 1, 0)
    s_loop(N_CHUNKS - 1, chunks.at[0], jnp.bool_(False))

    pltpu.make_async_copy(obufs.at[0], out_dst(0, 0), osems[0]).wait()
    pltpu.make_async_copy(obufs.at[1], out_dst(0, 0), osems[1]).wait()


def kernel(idx, table):
    idx_T = idx.astype(jnp.int32).T
    tableT = jnp.pad(table.T, ((0, 0), (0, DPAD - VOCAB)))
    out_p = _emb_transposed(idx_T, tableT)
    return jnp.transpose(out_p, (2, 0, 1))


# speedup vs baseline: 1.0189x; 1.0189x over previous
import functools

import jax
import jax.numpy as jnp
from jax import lax
from jax.experimental import pallas as pl
from jax.experimental.pallas import tpu as pltpu
from jax.experimental.pallas import tpu_sc as plsc

VOCAB = 1000
D = 1000
DPAD = 1024
BATCH = 4096
SEQ = 50
NW = 32
CD = 40
N_CHUNKS = D // CD
L = 16
NG = 128 // L
OBUF_BYTES = CD * 128 * 4

_mesh = plsc.VectorSubcoreMesh(core_axis_name="c", subcore_axis_name="s")


@functools.partial(
    pl.kernel,
    mesh=_mesh,
    out_type=jax.ShapeDtypeStruct((SEQ, D, BATCH), jnp.float32),
    compiler_params=pltpu.CompilerParams(use_tc_tiling_on_sc=True,
                                         needs_layout_passes=False),
    scratch_types=[
        pltpu.VMEM((SEQ, 128), jnp.int32),
        pltpu.VMEM((2, CD, DPAD), jnp.float32),
        pltpu.VMEM((2, CD, 128), jnp.float32),
        pltpu.SemaphoreType.DMA,
        pltpu.SemaphoreType.DMA,
        pltpu.SemaphoreType.DMA,
        pltpu.SemaphoreType.DMA,
    ],
)
def _emb_transposed(idx_hbm, tableT_hbm, out_hbm, idx_v, chunks, obufs,
                    g0, g1, o0, o1):
    gsems = [g0, g1]
    osems = [o0, o1]
    wid = lax.axis_index("s") * 2 + lax.axis_index("c")
    bcol = wid * 128
    pltpu.sync_copy(idx_hbm.at[:, pl.ds(bcol, 128)], idx_v)

    def chunk_src(c):
        return tableT_hbm.at[pl.ds(CD * c, CD), :]

    def out_dst(s, c):
        return out_hbm.at[s, pl.ds(CD * c, CD), pl.ds(bcol, 128)]

    def compute_tile(s, cbuf, obuf):
        cols = [idx_v[s, pl.ds(L * g, L)] for g in range(NG)]

        def drow(d, carry):
            dvec = jnp.full((L,), d, dtype=jnp.int32)
            for g in range(NG):
                obuf[d, pl.ds(L * g, L)] = plsc.load_gather(cbuf, [dvec, cols[g]])
            return carry

        lax.fori_loop(0, CD, drow, 0, unroll=4)

    def s_loop(c, cbuf, first):
        def s_pair(i, carry):
            for p in range(2):
                s = 2 * i + p

                @pl.when(jnp.logical_or(jnp.logical_not(first), i > 0))
                def _():
                    pltpu.make_async_copy(obufs.at[p], out_dst(0, 0),
                                          osems[p]).wait()
                compute_tile(s, cbuf, obufs.at[p])
                pltpu.async_copy(obufs.at[p], out_dst(s, c), osems[p])
            return carry

        lax.fori_loop(0, SEQ // 2, s_pair, 0)

    def wait_chunk(c, d):
        pltpu.make_async_copy(chunk_src(0), chunks.at[d], gsems[d]).wait()

    pltpu.async_copy(chunk_src(0), chunks.at[0], gsems[0])

    def chunk_pair(j, carry):
        c0 = 2 * j
        pltpu.async_copy(chunk_src(c0 + 1), chunks.at[1], gsems[1])
        wait_chunk(c0, 0)
        s_loop(c0, chunks.at[0], j == 0)
        pltpu.async_copy(chunk_src(c0 + 2), chunks.at[0], gsems[0])
        wait_chunk(c0 + 1, 1)
        s_loop(c0 + 1, chunks.at[1], jnp.bool_(False))
        return carry

    lax.fori_loop(0, N_CHUNKS // 2, chunk_pair, 0)
    wait_chunk(N_CHUNKS - 1, 0)
    s_loop(N_CHUNKS - 1, chunks.at[0], jnp.bool_(False))

    pltpu.make_async_copy(obufs.at[0], out_dst(0, 0), osems[0]).wait()
    pltpu.make_async_copy(obufs.at[1], out_dst(0, 0), osems[1]).wait()


def kernel(idx, table):
    idx_T = idx.astype(jnp.int32).T
    tableT = jnp.pad(table.T, ((0, 0), (0, DPAD - VOCAB)))
    out_p = _emb_transposed(idx_T, tableT)
    return jnp.transpose(out_p, (2, 0, 1))
